# baseline (speedup 1.0000x reference)
import functools
import math

import jax
import jax.numpy as jnp
from jax import lax
from jax.experimental import pallas as pl
from jax.experimental.pallas import tpu as pltpu
from jax.experimental.pallas import tpu_sc as plsc

_DIM = 64
_SCALE = math.sqrt(_DIM)
_LANES = 16
_CHUNK = 256


@functools.lru_cache(maxsize=None)
def _build_prep(vocab: int, dim: int):
    info = plsc.get_sparse_core_info()
    nw = info.num_cores * info.num_subcores
    n_blocks = vocab // 128
    per_w = n_blocks // nw
    n_rest = n_blocks - per_w * nw
    out_rows = vocab // 2
    mesh = plsc.VectorSubcoreMesh(core_axis_name="c", subcore_axis_name="s")

    @functools.partial(
        pl.kernel,
        mesh=mesh,
        compiler_params=pltpu.CompilerParams(needs_layout_passes=False),
        out_type=jax.ShapeDtypeStruct((out_rows, 128), jnp.float32),
        scratch_types=[
            pltpu.VMEM((dim, 128), jnp.float32),
            pltpu.VMEM((dim, 128), jnp.float32),
            pltpu.VMEM((64, 128), jnp.float32),
            pltpu.VMEM((64, 128), jnp.float32),
            pltpu.SemaphoreType.DMA,
            pltpu.SemaphoreType.DMA,
            pltpu.SemaphoreType.DMA,
            pltpu.SemaphoreType.DMA,
        ],
    )
    def sc_prep(tabt_hbm, out_hbm, v0, v1, o0, o1, gs0, gs1, ss0, ss1):
        wid = lax.axis_index("s") * info.num_cores + lax.axis_index("c")
        vins = (v0, v1)
        vouts = (o0, o1)
        gsems = (gs0, gs1)
        ssems = (ss0, ss1)
        lane = lax.iota(jnp.int32, _LANES)

        def start_read(c, p):
            pltpu.async_copy(
                tabt_hbm.at[:, pl.ds(c * 128, 128)],
                vins[p].at[:, pl.ds(0, 128)],
                gsems[p],
            )

        def wait_read(c, p):
            pltpu.make_async_copy(
                tabt_hbm.at[:, pl.ds(c * 128, 128)],
                vins[p].at[:, pl.ds(0, 128)],
                gsems[p],
            ).wait()

        lane_half = lane // 2
        lane_par64 = (lane % 2) * dim

        def transpose(p):
            vin = vins[p]
            vout = vouts[p]

            @plsc.parallel_loop(0, 8, unroll=2)
            def per_gj(gj):
                vrow = 8 * gj + lane_half
                vcol = 16 * gj + lane
                for blk in range(0, dim, 16):
                    loaded = []
                    for d0 in range(blk, blk + 16):
                        dd = jnp.bitwise_and(lane + d0, dim - 1)
                        loaded.append(
                            (dd, plsc.load_gather(vin, [dd, vcol]))
                        )
                    for dd, vals in loaded:
                        plsc.store_scatter(
                            vout, [vrow, lane_par64 + dd], vals * _SCALE
                        )

        def start_write(c, p):
            pltpu.async_copy(
                vouts[p], out_hbm.at[pl.ds(c * 64, 64), :], ssems[p]
            )

        def wait_write(c, p):
            pltpu.make_async_copy(
                vouts[p], out_hbm.at[pl.ds(c * 64, 64), :], ssems[p]
            ).wait()

        start_read(wid, 0)

        def pair(kk, carry):
            c0 = (2 * kk) * nw + wid
            c1 = c0 + nw
            c2 = c1 + nw
            start_read(c1, 1)
            wait_read(c0, 0)
            pl.when(kk > 0)(lambda: wait_write(c0 - 2 * nw, 0))
            transpose(0)
            start_write(c0, 0)
            pl.when(kk < per_w // 2 - 1)(lambda: start_read(c2, 0))
            wait_read(c1, 1)
            pl.when(kk > 0)(lambda: wait_write(c1 - 2 * nw, 1))
            transpose(1)
            start_write(c1, 1)
            return carry

        lax.fori_loop(0, per_w // 2, pair, 0)
        wait_write((per_w - 2) * nw + wid, 0)
        wait_write((per_w - 1) * nw + wid, 1)

        @pl.when(wid < n_rest)
        def _():
            c = per_w * nw + wid
            start_read(c, 0)
            wait_read(c, 0)
            transpose(0)
            start_write(c, 0)
            wait_write(c, 0)

    return sc_prep


@functools.lru_cache(maxsize=None)
def _build_tail(vocab: int, dim: int):
    n_blocks = vocab // 128
    tail = vocab - n_blocks * 128
    out_rows = vocab // 2

    def body(t_ref, tabc_ref, o_ref):
        del tabc_ref
        tt = t_ref[...].T * _SCALE
        tt3 = tt.reshape(64, 2, dim)
        a = tt3[: tail // 2, 0, :]
        b = tt3[: tail // 2, 1, :]
        o_ref[...] = jnp.concatenate([a, b], axis=1)

    return pl.pallas_call(
        body,
        grid=(1,),
        in_specs=[
            pl.BlockSpec((dim, 128), lambda i: (0, n_blocks)),
            pl.BlockSpec((8, 128), lambda i: (0, 0)),
        ],
        out_specs=pl.BlockSpec(
            (tail // 2, 128), lambda i: (n_blocks * 64 // (tail // 2), 0)
        ),
        out_shape=jax.ShapeDtypeStruct((out_rows, 128), jnp.float32),
        input_output_aliases={1: 0},
    )


@functools.lru_cache(maxsize=None)
def _build_gather(seq: int, batch: int, dim: int):
    info = plsc.get_sparse_core_info()
    nw = info.num_cores * info.num_subcores
    chunks_per_s = batch // _CHUNK
    n_units = seq * chunks_per_s
    assert n_units % nw == 0
    units_per_w = n_units // nw

    mesh = plsc.VectorSubcoreMesh(core_axis_name="c", subcore_axis_name="s")

    @functools.partial(
        pl.kernel,
        mesh=mesh,
        compiler_params=pltpu.CompilerParams(needs_layout_passes=False),
        out_type=jax.ShapeDtypeStruct((seq, dim, batch), jnp.float32),
        scratch_types=[
            pltpu.VMEM((_CHUNK,), jnp.int32),
            pltpu.VMEM((_CHUNK,), jnp.int32),
            pltpu.VMEM((_CHUNK,), jnp.int32),
            pltpu.VMEM((_CHUNK,), jnp.int32),
            pltpu.VMEM((_CHUNK, 128), jnp.float32),
            pltpu.VMEM((_CHUNK, 128), jnp.float32),
            pltpu.VMEM((dim, _CHUNK), jnp.float32),
            pltpu.VMEM((dim, _CHUNK), jnp.float32),
            pltpu.SemaphoreType.DMA,
            pltpu.SemaphoreType.DMA,
            pltpu.SemaphoreType.DMA,
            pltpu.SemaphoreType.DMA,
        ],
    )
    def sc_embed(xt_hbm, tab_hbm, out_hbm, iraw0, iraw1, ih0, ih1, g0, g1,
                 o0, o1, gs0, gs1, ss0, ss1):
        wid = lax.axis_index("s") * info.num_cores + lax.axis_index("c")
        u_base = wid * units_per_w
        iraws = (iraw0, iraw1)
        ihalfs = (ih0, ih1)
        gbufs = (g0, g1)
        obufs = (o0, o1)
        gsems = (gs0, gs1)
        ssems = (ss0, ss1)

        lane = lax.iota(jnp.int32, _LANES)

        def unit_pos(k):
            u = u_base + k
            return u // chunks_per_s, (u % chunks_per_s) * _CHUNK

        def start_gather(k, p):
            s, b0 = unit_pos(k)
            pltpu.sync_copy(xt_hbm.at[s, pl.ds(b0, _CHUNK)], iraws[p])
            def halve(j, c):
                sl = pl.ds(j * _LANES, _LANES)
                ihalfs[p][sl] = lax.shift_right_logical(iraws[p][sl], 1)
                return c

            lax.fori_loop(0, _CHUNK // _LANES, halve, 0)
            pltpu.async_copy(tab_hbm.at[ihalfs[p]], gbufs[p], gsems[p])

        def wait_gather(p):
            pltpu.make_async_copy(
                tab_hbm.at[ihalfs[p]], gbufs[p], gsems[p]
            ).wait()

        def transpose_block(p):
            g = gbufs[p]
            o = obufs[p]
            ir = iraws[p]

            @plsc.parallel_loop(0, _CHUNK // _LANES, unroll=2)
            def col_group(gi):
                sl = pl.ds(gi * _LANES, _LANES)
                rows = gi * _LANES + lane
                off = lax.shift_left(jnp.bitwise_and(ir[sl], 1), 6)
                for blk in range(0, dim, 16):
                    loaded = []
                    for d0 in range(blk, blk + 16):
                        dd = jnp.bitwise_and(lane + d0, dim - 1)
                        loaded.append(
                            (dd, plsc.load_gather(g, [rows, off + dd]))
                        )
                    for dd, vals in loaded:
                        plsc.store_scatter(o, [dd, rows], vals)

        def start_store(k, p):
            s, b0 = unit_pos(k)
            pltpu.async_copy(
                obufs[p], out_hbm.at[s, :, pl.ds(b0, _CHUNK)], ssems[p]
            )

        def wait_store(k, p):
            s, b0 = unit_pos(k)
            pltpu.make_async_copy(
                obufs[p], out_hbm.at[s, :, pl.ds(b0, _CHUNK)], ssems[p]
            ).wait()

        n_pairs = units_per_w // 2
        assert units_per_w % 2 == 0 and n_pairs >= 2

        start_gather(0, 0)

        def pair(kk, carry):
            k0 = 2 * kk
            start_gather(k0 + 1, 1)
            wait_gather(0)
            pl.when(kk > 0)(lambda: wait_store(k0, 0))
            transpose_block(0)
            start_store(k0, 0)
            pl.when(kk < n_pairs - 1)(lambda: start_gather(k0 + 2, 0))
            wait_gather(1)
            pl.when(kk > 0)(lambda: wait_store(k0 + 1, 1))
            transpose_block(1)
            start_store(k0 + 1, 1)
            return carry

        lax.fori_loop(0, n_pairs, pair, 0)
        wait_store(units_per_w - 2, 0)
        wait_store(units_per_w - 1, 1)

    return sc_embed


def kernel(x, table):
    b, s = x.shape
    vocab, dim = table.shape
    tab_t = table.T
    tab_c = _build_prep(vocab, dim)(tab_t)
    tab_c = _build_tail(vocab, dim)(tab_t, tab_c)
    out_t = _build_gather(s, b, dim)(x.T.astype(jnp.int32), tab_c)
    return out_t.transpose(2, 0, 1)


# speedup vs baseline: 1.2785x; 1.2785x over previous
import functools
import math

import jax
import jax.numpy as jnp
from jax import lax
from jax.experimental import pallas as pl
from jax.experimental.pallas import tpu as pltpu
from jax.experimental.pallas import tpu_sc as plsc

_DIM = 64
_SCALE = math.sqrt(_DIM)
_LANES = 16
_CHUNK = 256


@functools.lru_cache(maxsize=None)
def _build_prep(vocab: int, dim: int):
    info = plsc.get_sparse_core_info()
    nw = info.num_cores * info.num_subcores
    n_blocks = vocab // 128
    per_w = n_blocks // nw
    n_rest = n_blocks - per_w * nw
    out_rows = vocab // 2
    mesh = plsc.VectorSubcoreMesh(core_axis_name="c", subcore_axis_name="s")

    @functools.partial(
        pl.kernel,
        mesh=mesh,
        compiler_params=pltpu.CompilerParams(needs_layout_passes=False),
        out_type=jax.ShapeDtypeStruct((out_rows, 128), jnp.float32),
        scratch_types=[
            pltpu.VMEM((dim, 128), jnp.float32),
            pltpu.VMEM((dim, 128), jnp.float32),
            pltpu.VMEM((64, 128), jnp.float32),
            pltpu.VMEM((64, 128), jnp.float32),
            pltpu.SemaphoreType.DMA,
            pltpu.SemaphoreType.DMA,
            pltpu.SemaphoreType.DMA,
            pltpu.SemaphoreType.DMA,
        ],
    )
    def sc_prep(tabt_hbm, out_hbm, v0, v1, o0, o1, gs0, gs1, ss0, ss1):
        wid = lax.axis_index("s") * info.num_cores + lax.axis_index("c")
        vins = (v0, v1)
        vouts = (o0, o1)
        gsems = (gs0, gs1)
        ssems = (ss0, ss1)
        lane = lax.iota(jnp.int32, _LANES)

        def start_read(c, p):
            pltpu.async_copy(
                tabt_hbm.at[:, pl.ds(c * 128, 128)],
                vins[p].at[:, pl.ds(0, 128)],
                gsems[p],
            )

        def wait_read(c, p):
            pltpu.make_async_copy(
                tabt_hbm.at[:, pl.ds(c * 128, 128)],
                vins[p].at[:, pl.ds(0, 128)],
                gsems[p],
            ).wait()

        lane_half = lane // 2
        lane_par64 = (lane % 2) * dim

        def transpose(p):
            vin = vins[p]
            vout = vouts[p]

            @plsc.parallel_loop(0, 8, unroll=2)
            def per_gj(gj):
                vrow = 8 * gj + lane_half
                vcol_base = lane_par64
                for d0 in range(dim):
                    dd = jnp.bitwise_and(lane + d0, dim - 1)
                    vals = plsc.load_gather(vin, [dd, 16 * gj + lane])
                    plsc.store_scatter(
                        vout, [vrow, vcol_base + dd], vals * _SCALE
                    )

        def start_write(c, p):
            pltpu.async_copy(
                vouts[p], out_hbm.at[pl.ds(c * 64, 64), :], ssems[p]
            )

        def wait_write(c, p):
            pltpu.make_async_copy(
                vouts[p], out_hbm.at[pl.ds(c * 64, 64), :], ssems[p]
            ).wait()

        start_read(wid, 0)

        def pair(kk, carry):
            c0 = (2 * kk) * nw + wid
            c1 = c0 + nw
            c2 = c1 + nw
            start_read(c1, 1)
            wait_read(c0, 0)
            pl.when(kk > 0)(lambda: wait_write(c0 - 2 * nw, 0))
            transpose(0)
            start_write(c0, 0)
            pl.when(kk < per_w // 2 - 1)(lambda: start_read(c2, 0))
            wait_read(c1, 1)
            pl.when(kk > 0)(lambda: wait_write(c1 - 2 * nw, 1))
            transpose(1)
            start_write(c1, 1)
            return carry

        lax.fori_loop(0, per_w // 2, pair, 0)
        wait_write((per_w - 2) * nw + wid, 0)
        wait_write((per_w - 1) * nw + wid, 1)

        @pl.when(wid < n_rest)
        def _():
            c = per_w * nw + wid
            start_read(c, 0)
            wait_read(c, 0)
            transpose(0)
            start_write(c, 0)
            wait_write(c, 0)

    return sc_prep


@functools.lru_cache(maxsize=None)
def _build_tail(vocab: int, dim: int):
    n_blocks = vocab // 128
    tail = vocab - n_blocks * 128
    out_rows = vocab // 2

    def body(t_ref, tabc_ref, o_ref):
        del tabc_ref
        tt = t_ref[...].T * _SCALE
        tt3 = tt.reshape(64, 2, dim)
        a = tt3[: tail // 2, 0, :]
        b = tt3[: tail // 2, 1, :]
        o_ref[...] = jnp.concatenate([a, b], axis=1)

    return pl.pallas_call(
        body,
        grid=(1,),
        in_specs=[
            pl.BlockSpec((dim, 128), lambda i: (0, n_blocks)),
            pl.BlockSpec((8, 128), lambda i: (0, 0)),
        ],
        out_specs=pl.BlockSpec(
            (tail // 2, 128), lambda i: (n_blocks * 64 // (tail // 2), 0)
        ),
        out_shape=jax.ShapeDtypeStruct((out_rows, 128), jnp.float32),
        input_output_aliases={1: 0},
    )


@functools.lru_cache(maxsize=None)
def _build_gather(seq: int, batch: int, dim: int):
    info = plsc.get_sparse_core_info()
    nw = info.num_cores * info.num_subcores
    chunks_per_s = batch // _CHUNK
    n_units = seq * chunks_per_s
    assert n_units % nw == 0
    units_per_w = n_units // nw

    mesh = plsc.VectorSubcoreMesh(core_axis_name="c", subcore_axis_name="s")

    @functools.partial(
        pl.kernel,
        mesh=mesh,
        compiler_params=pltpu.CompilerParams(needs_layout_passes=False),
        out_type=jax.ShapeDtypeStruct((seq, dim, batch), jnp.float32),
        scratch_types=[
            pltpu.VMEM((_CHUNK,), jnp.int32),
            pltpu.VMEM((_CHUNK,), jnp.int32),
            pltpu.VMEM((_CHUNK,), jnp.int32),
            pltpu.VMEM((_CHUNK,), jnp.int32),
            pltpu.VMEM((_CHUNK, 128), jnp.float32),
            pltpu.VMEM((_CHUNK, 128), jnp.float32),
            pltpu.VMEM((dim, _CHUNK), jnp.float32),
            pltpu.VMEM((dim, _CHUNK), jnp.float32),
            pltpu.SemaphoreType.DMA,
            pltpu.SemaphoreType.DMA,
            pltpu.SemaphoreType.DMA,
            pltpu.SemaphoreType.DMA,
        ],
    )
    def sc_embed(xt_hbm, tab_hbm, out_hbm, iraw0, iraw1, ih0, ih1, g0, g1,
                 o0, o1, gs0, gs1, ss0, ss1):
        wid = lax.axis_index("s") * info.num_cores + lax.axis_index("c")
        u_base = wid * units_per_w
        iraws = (iraw0, iraw1)
        ihalfs = (ih0, ih1)
        gbufs = (g0, g1)
        obufs = (o0, o1)
        gsems = (gs0, gs1)
        ssems = (ss0, ss1)

        lane = lax.iota(jnp.int32, _LANES)

        def unit_pos(k):
            u = u_base + k
            return u // chunks_per_s, (u % chunks_per_s) * _CHUNK

        def start_gather(k, p):
            s, b0 = unit_pos(k)
            pltpu.sync_copy(xt_hbm.at[s, pl.ds(b0, _CHUNK)], iraws[p])
            def halve(j, c):
                sl = pl.ds(j * _LANES, _LANES)
                ihalfs[p][sl] = lax.shift_right_logical(iraws[p][sl], 1)
                return c

            lax.fori_loop(0, _CHUNK // _LANES, halve, 0)
            pltpu.async_copy(tab_hbm.at[ihalfs[p]], gbufs[p], gsems[p])

        def wait_gather(p):
            pltpu.make_async_copy(
                tab_hbm.at[ihalfs[p]], gbufs[p], gsems[p]
            ).wait()

        def transpose_block(p):
            g = gbufs[p]
            o = obufs[p]
            ir = iraws[p]

            @plsc.parallel_loop(0, _CHUNK // _LANES, unroll=2)
            def col_group(gi):
                sl = pl.ds(gi * _LANES, _LANES)
                rows = gi * _LANES + lane
                off = lax.shift_left(jnp.bitwise_and(ir[sl], 1), 6)
                for d0 in range(dim):
                    dd = jnp.bitwise_and(lane + d0, dim - 1)
                    vals = plsc.load_gather(g, [rows, off + dd])
                    plsc.store_scatter(o, [dd, rows], vals)

        def start_store(k, p):
            s, b0 = unit_pos(k)
            pltpu.async_copy(
                obufs[p], out_hbm.at[s, :, pl.ds(b0, _CHUNK)], ssems[p]
            )

        def wait_store(k, p):
            s, b0 = unit_pos(k)
            pltpu.make_async_copy(
                obufs[p], out_hbm.at[s, :, pl.ds(b0, _CHUNK)], ssems[p]
            ).wait()

        n_pairs = units_per_w // 2
        assert units_per_w % 2 == 0 and n_pairs >= 2

        start_gather(0, 0)

        def pair(kk, carry):
            k0 = 2 * kk
            start_gather(k0 + 1, 1)
            wait_gather(0)
            pl.when(kk > 0)(lambda: wait_store(k0, 0))
            transpose_block(0)
            start_store(k0, 0)
            pl.when(kk < n_pairs - 1)(lambda: start_gather(k0 + 2, 0))
            wait_gather(1)
            pl.when(kk > 0)(lambda: wait_store(k0 + 1, 1))
            transpose_block(1)
            start_store(k0 + 1, 1)
            return carry

        lax.fori_loop(0, n_pairs, pair, 0)
        wait_store(units_per_w - 2, 0)
        wait_store(units_per_w - 1, 1)

    return sc_embed


def kernel(x, table):
    b, s = x.shape
    vocab, dim = table.shape
    tab_t = table.T
    tab_c = _build_prep(vocab, dim)(tab_t)
    tab_c = _build_tail(vocab, dim)(tab_t, tab_c)
    out_t = _build_gather(s, b, dim)(x.T.astype(jnp.int32), tab_c)
    return out_t.transpose(2, 0, 1)


# speedup vs baseline: 3.0957x; 2.4214x over previous
import functools
import math

import jax
import jax.numpy as jnp
from jax import lax
from jax.experimental import pallas as pl
from jax.experimental.pallas import tpu as pltpu
from jax.experimental.pallas import tpu_sc as plsc

_DIM = 64
_SCALE = math.sqrt(_DIM)
_LANES = 16
_CHUNK = 256


@functools.lru_cache(maxsize=None)
def _build_prep(vocab: int, dim: int):
    info = plsc.get_sparse_core_info()
    nw = info.num_cores * info.num_subcores
    n_blocks = vocab // 128
    per_w = n_blocks // nw
    n_rest = n_blocks - per_w * nw
    out_rows = vocab // 2
    mesh = plsc.VectorSubcoreMesh(core_axis_name="c", subcore_axis_name="s")

    @functools.partial(
        pl.kernel,
        mesh=mesh,
        compiler_params=pltpu.CompilerParams(needs_layout_passes=False),
        out_type=jax.ShapeDtypeStruct((out_rows, 128), jnp.float32),
        scratch_types=[
            pltpu.VMEM((dim, 128), jnp.float32),
            pltpu.VMEM((dim, 128), jnp.float32),
            pltpu.VMEM((64, 128), jnp.float32),
            pltpu.VMEM((64, 128), jnp.float32),
            pltpu.SemaphoreType.DMA,
            pltpu.SemaphoreType.DMA,
            pltpu.SemaphoreType.DMA,
            pltpu.SemaphoreType.DMA,
        ],
    )
    def sc_prep(tabt_hbm, out_hbm, v0, v1, o0, o1, gs0, gs1, ss0, ss1):
        wid = lax.axis_index("s") * info.num_cores + lax.axis_index("c")
        vins = (v0, v1)
        vouts = (o0, o1)
        gsems = (gs0, gs1)
        ssems = (ss0, ss1)
        lane = lax.iota(jnp.int32, _LANES)

        def start_read(c, p):
            pltpu.async_copy(
                tabt_hbm.at[:, pl.ds(c * 128, 128)],
                vins[p].at[:, pl.ds(0, 128)],
                gsems[p],
            )

        def wait_read(c, p):
            pltpu.make_async_copy(
                tabt_hbm.at[:, pl.ds(c * 128, 128)],
                vins[p].at[:, pl.ds(0, 128)],
                gsems[p],
            ).wait()

        lane_half = lane // 2
        lane_par64 = (lane % 2) * dim

        def transpose(p):
            vin = vins[p]
            vout = vouts[p]

            @plsc.parallel_loop(0, 8, unroll=4)
            def per_gj(gj):
                vrow = 8 * gj + lane_half
                vcol_base = lane_par64
                for d0 in range(dim):
                    dd = jnp.bitwise_and(lane + d0, dim - 1)
                    vals = plsc.load_gather(vin, [dd, 16 * gj + lane])
                    plsc.store_scatter(
                        vout, [vrow, vcol_base + dd], vals * _SCALE
                    )

        def start_write(c, p):
            pltpu.async_copy(
                vouts[p], out_hbm.at[pl.ds(c * 64, 64), :], ssems[p]
            )

        def wait_write(c, p):
            pltpu.make_async_copy(
                vouts[p], out_hbm.at[pl.ds(c * 64, 64), :], ssems[p]
            ).wait()

        start_read(wid, 0)

        def pair(kk, carry):
            c0 = (2 * kk) * nw + wid
            c1 = c0 + nw
            c2 = c1 + nw
            start_read(c1, 1)
            wait_read(c0, 0)
            pl.when(kk > 0)(lambda: wait_write(c0 - 2 * nw, 0))
            transpose(0)
            start_write(c0, 0)
            pl.when(kk < per_w // 2 - 1)(lambda: start_read(c2, 0))
            wait_read(c1, 1)
            pl.when(kk > 0)(lambda: wait_write(c1 - 2 * nw, 1))
            transpose(1)
            start_write(c1, 1)
            return carry

        lax.fori_loop(0, per_w // 2, pair, 0)
        wait_write((per_w - 2) * nw + wid, 0)
        wait_write((per_w - 1) * nw + wid, 1)

        @pl.when(wid < n_rest)
        def _():
            c = per_w * nw + wid
            start_read(c, 0)
            wait_read(c, 0)
            transpose(0)
            start_write(c, 0)
            wait_write(c, 0)

    return sc_prep


@functools.lru_cache(maxsize=None)
def _build_tail(vocab: int, dim: int):
    n_blocks = vocab // 128
    tail = vocab - n_blocks * 128
    out_rows = vocab // 2

    def body(t_ref, tabc_ref, o_ref):
        del tabc_ref
        tt = t_ref[...].T * _SCALE
        tt3 = tt.reshape(64, 2, dim)
        a = tt3[: tail // 2, 0, :]
        b = tt3[: tail // 2, 1, :]
        o_ref[...] = jnp.concatenate([a, b], axis=1)

    return pl.pallas_call(
        body,
        grid=(1,),
        in_specs=[
            pl.BlockSpec((dim, 128), lambda i: (0, n_blocks)),
            pl.BlockSpec((8, 128), lambda i: (0, 0)),
        ],
        out_specs=pl.BlockSpec(
            (tail // 2, 128), lambda i: (n_blocks * 64 // (tail // 2), 0)
        ),
        out_shape=jax.ShapeDtypeStruct((out_rows, 128), jnp.float32),
        input_output_aliases={1: 0},
    )


@functools.lru_cache(maxsize=None)
def _build_gather(seq: int, batch: int, dim: int):
    info = plsc.get_sparse_core_info()
    nw = info.num_cores * info.num_subcores
    chunks_per_s = batch // _CHUNK
    n_units = seq * chunks_per_s
    assert n_units % nw == 0
    units_per_w = n_units // nw

    mesh = plsc.VectorSubcoreMesh(core_axis_name="c", subcore_axis_name="s")

    @functools.partial(
        pl.kernel,
        mesh=mesh,
        compiler_params=pltpu.CompilerParams(needs_layout_passes=False),
        out_type=jax.ShapeDtypeStruct((seq, dim, batch), jnp.float32),
        scratch_types=[
            pltpu.VMEM((_CHUNK,), jnp.int32),
            pltpu.VMEM((_CHUNK,), jnp.int32),
            pltpu.VMEM((_CHUNK,), jnp.int32),
            pltpu.VMEM((_CHUNK,), jnp.int32),
            pltpu.VMEM((_CHUNK, 128), jnp.float32),
            pltpu.VMEM((_CHUNK, 128), jnp.float32),
            pltpu.VMEM((dim, _CHUNK), jnp.float32),
            pltpu.VMEM((dim, _CHUNK), jnp.float32),
            pltpu.SemaphoreType.DMA,
            pltpu.SemaphoreType.DMA,
            pltpu.SemaphoreType.DMA,
            pltpu.SemaphoreType.DMA,
        ],
    )
    def sc_embed(xt_hbm, tab_hbm, out_hbm, iraw0, iraw1, ih0, ih1, g0, g1,
                 o0, o1, gs0, gs1, ss0, ss1):
        wid = lax.axis_index("s") * info.num_cores + lax.axis_index("c")
        u_base = wid * units_per_w
        iraws = (iraw0, iraw1)
        ihalfs = (ih0, ih1)
        gbufs = (g0, g1)
        obufs = (o0, o1)
        gsems = (gs0, gs1)
        ssems = (ss0, ss1)

        lane = lax.iota(jnp.int32, _LANES)

        def unit_pos(k):
            u = u_base + k
            return u // chunks_per_s, (u % chunks_per_s) * _CHUNK

        def start_gather(k, p):
            s, b0 = unit_pos(k)
            pltpu.sync_copy(xt_hbm.at[s, pl.ds(b0, _CHUNK)], iraws[p])
            def halve(j, c):
                sl = pl.ds(j * _LANES, _LANES)
                ihalfs[p][sl] = lax.shift_right_logical(iraws[p][sl], 1)
                return c

            lax.fori_loop(0, _CHUNK // _LANES, halve, 0)
            pltpu.async_copy(tab_hbm.at[ihalfs[p]], gbufs[p], gsems[p])

        def wait_gather(p):
            pltpu.make_async_copy(
                tab_hbm.at[ihalfs[p]], gbufs[p], gsems[p]
            ).wait()

        def transpose_block(p):
            g = gbufs[p]
            o = obufs[p]
            ir = iraws[p]

            @plsc.parallel_loop(0, _CHUNK // _LANES, unroll=4)
            def col_group(gi):
                sl = pl.ds(gi * _LANES, _LANES)
                rows = gi * _LANES + lane
                off = lax.shift_left(jnp.bitwise_and(ir[sl], 1), 6)
                for d0 in range(dim):
                    dd = jnp.bitwise_and(lane + d0, dim - 1)
                    vals = plsc.load_gather(g, [rows, off + dd])
                    plsc.store_scatter(o, [dd, rows], vals)

        def start_store(k, p):
            s, b0 = unit_pos(k)
            pltpu.async_copy(
                obufs[p], out_hbm.at[s, :, pl.ds(b0, _CHUNK)], ssems[p]
            )

        def wait_store(k, p):
            s, b0 = unit_pos(k)
            pltpu.make_async_copy(
                obufs[p], out_hbm.at[s, :, pl.ds(b0, _CHUNK)], ssems[p]
            ).wait()

        n_pairs = units_per_w // 2
        assert units_per_w % 2 == 0 and n_pairs >= 2

        start_gather(0, 0)

        def pair(kk, carry):
            k0 = 2 * kk
            start_gather(k0 + 1, 1)
            wait_gather(0)
            pl.when(kk > 0)(lambda: wait_store(k0, 0))
            transpose_block(0)
            start_store(k0, 0)
            pl.when(kk < n_pairs - 1)(lambda: start_gather(k0 + 2, 0))
            wait_gather(1)
            pl.when(kk > 0)(lambda: wait_store(k0 + 1, 1))
            transpose_block(1)
            start_store(k0 + 1, 1)
            return carry

        lax.fori_loop(0, n_pairs, pair, 0)
        wait_store(units_per_w - 2, 0)
        wait_store(units_per_w - 1, 1)

    return sc_embed


def kernel(x, table):
    b, s = x.shape
    vocab, dim = table.shape
    tab_t = table.T
    tab_c = _build_prep(vocab, dim)(tab_t)
    tab_c = _build_tail(vocab, dim)(tab_t, tab_c)
    out_t = _build_gather(s, b, dim)(x.T.astype(jnp.int32), tab_c)
    return out_t.transpose(2, 0, 1)
